# baseline (speedup 1.0000x reference)
import functools

import jax
import jax.numpy as jnp
from jax import lax
from jax.experimental import pallas as pl
from jax.experimental.pallas import tpu as pltpu
from jax.experimental.pallas import tpu_sc as plsc

NC = 2
NS = 16
L = 16
NW = NC * NS
CA = 128
CB = 256
ROW_BLK = 2000


def _cdiv(a, b):
  return (a + b - 1) // b


def _prep_body(x_ref, w_ref, asrc_ref, adst_ref, h_ref, os_ref, od_ref):
  xb = x_ref[...]
  h = jnp.dot(xb, w_ref[...], preferred_element_type=jnp.float32,
              precision=lax.Precision.HIGHEST)
  h_ref[...] = h
  os_ref[...] = jnp.dot(h, asrc_ref[...], preferred_element_type=jnp.float32,
                        precision=lax.Precision.HIGHEST)
  od_ref[...] = jnp.dot(h, adst_ref[...], preferred_element_type=jnp.float32,
                        precision=lax.Precision.HIGHEST)


def _tc_prep(x, W, Asrc, Adst):
  N, D = x.shape
  grid = (N // ROW_BLK,)
  return pl.pallas_call(
      _prep_body,
      grid=grid,
      in_specs=[
          pl.BlockSpec((ROW_BLK, D), lambda i: (i, 0)),
          pl.BlockSpec(W.shape, lambda i: (0, 0)),
          pl.BlockSpec(Asrc.shape, lambda i: (0, 0)),
          pl.BlockSpec(Adst.shape, lambda i: (0, 0)),
      ],
      out_specs=[
          pl.BlockSpec((ROW_BLK, W.shape[1]), lambda i: (i, 0)),
          pl.BlockSpec((ROW_BLK, L), lambda i: (i, 0)),
          pl.BlockSpec((ROW_BLK, L), lambda i: (i, 0)),
      ],
      out_shape=[
          jax.ShapeDtypeStruct((N, W.shape[1]), jnp.float32),
          jax.ShapeDtypeStruct((N, L), jnp.float32),
          jax.ShapeDtypeStruct((N, L), jnp.float32),
      ],
  )(x, W, Asrc, Adst)


def _invden_body(p_ref, e_ref, inv_ref):
  s = jnp.sum(p_ref[...], axis=0)
  inv = 1.0 / (s + 1e-16)
  inv_ref[...] = jnp.dot(inv, e_ref[...], preferred_element_type=jnp.float32,
                         precision=lax.Precision.HIGHEST)


def _tc_invexp(parts, expand):
  _, R, H = parts.shape
  BR = 640
  grid = (R // BR,)
  return pl.pallas_call(
      _invden_body,
      grid=grid,
      in_specs=[
          pl.BlockSpec((NW, BR, H), lambda i: (0, i, 0)),
          pl.BlockSpec(expand.shape, lambda i: (0, 0)),
      ],
      out_specs=pl.BlockSpec((BR, 128), lambda i: (i, 0)),
      out_shape=jax.ShapeDtypeStruct((R, 128), jnp.float32),
  )(parts, expand)


def _post_body(p0_ref, p1_ref, inv_ref, res_ref, b_ref, g_ref, be_ref, o_ref,
               *, act):
  s = (p0_ref[...] + p1_ref[...]) * inv_ref[...] + b_ref[...]
  mu = jnp.mean(s, axis=-1, keepdims=True)
  var = jnp.mean((s - mu) ** 2, axis=-1, keepdims=True)
  y = (s - mu) / jnp.sqrt(var + 1e-5) * g_ref[...] + be_ref[...]
  if act:
    y = jnp.where(y >= 0, y, 0.2 * y) + res_ref[...]
  o_ref[...] = y


def _tc_post(p0, p1, inv, res, b, g, be, act):
  N, D = p0.shape
  grid = (N // ROW_BLK,)
  vec = pl.BlockSpec((1, D), lambda i: (0, 0))
  return pl.pallas_call(
      functools.partial(_post_body, act=act),
      grid=grid,
      in_specs=[
          pl.BlockSpec((ROW_BLK, D), lambda i: (i, 0)),
          pl.BlockSpec((ROW_BLK, D), lambda i: (i, 0)),
          pl.BlockSpec((ROW_BLK, D), lambda i: (i, 0)),
          pl.BlockSpec((ROW_BLK, D), lambda i: (i, 0)),
          vec, vec, vec,
      ],
      out_specs=pl.BlockSpec((ROW_BLK, D), lambda i: (i, 0)),
      out_shape=jax.ShapeDtypeStruct((N, D), jnp.float32),
  )(p0, p1, inv, res, b.reshape(1, D), g.reshape(1, D), be.reshape(1, D))


def _zero_vmem_rows(ref, nrows, ncols):
  nseg = ncols // L

  def body(k, c):
    for j in range(nseg):
      ref[k, pl.ds(j * L, L)] = jnp.zeros((L,), jnp.float32)
    return c

  lax.fori_loop(0, nrows, body, 0)


def _make_edge_stats(n_pad, e_pad):
  ept = e_pad // NW
  epc = ept // CA
  dn = n_pad * 8
  mesh = plsc.VectorSubcoreMesh(core_axis_name="c", subcore_axis_name="s")

  @functools.partial(
      pl.kernel,
      mesh=mesh,
      compiler_params=pltpu.CompilerParams(needs_layout_passes=False),
      out_type=[
          jax.ShapeDtypeStruct((e_pad * L,), jnp.float32),
          jax.ShapeDtypeStruct((NW, dn), jnp.float32),
      ],
      scratch_types=[
          pltpu.VMEM((5 * CA,), jnp.int32),
          pltpu.VMEM((CA, 128), jnp.float32),
          pltpu.VMEM((CA, 128), jnp.float32),
          pltpu.VMEM((CA * L,), jnp.float32),
          pltpu.VMEM((dn,), jnp.float32),
          pltpu.SemaphoreType.DMA,
      ],
  )
  def edge_stats(stats_src, stats_dst, pk,
                 wbuf, den_part,
                 pk_v, srows_v, drows_v, w_v,
                 denp_v, sem):
    cid = lax.axis_index("c")
    sid = lax.axis_index("s")
    wid = sid * NC + cid

    def zbody(i, c):
      denp_v[pl.ds(i * L, L)] = jnp.zeros((L,), jnp.float32)
      return c

    lax.fori_loop(0, dn // L, zbody, 0)

    iota16 = lax.iota(jnp.int32, 16)
    mask8 = iota16 < 8
    base = wid * ept

    def chunk_body(c, carry):
      off = base + c * CA
      pltpu.sync_copy(pk.at[pl.ds((wid * epc + c) * 5 * CA, 5 * CA)], pk_v)
      d1 = pltpu.async_copy(stats_src.at[pk_v.at[pl.ds(0, CA)]], srows_v, sem)
      d2 = pltpu.async_copy(stats_dst.at[pk_v.at[pl.ds(CA, CA)]], drows_v, sem)
      d1.wait()
      d2.wait()

      def q_body(q, c2):
        sov = pk_v[pl.ds(2 * CA + q * L, L)]
        dov = pk_v[pl.ds(3 * CA + q * L, L)]
        dfv = pk_v[pl.ds(4 * CA + q * L, L)]
        for j in range(L):
          k = q * L + j
          sv = srows_v[k, pl.ds(sov[j], L)]
          dv = drows_v[k, pl.ds(dov[j], L)]
          e = sv + dv
          e = jnp.where(e >= 0, e, e * 0.2)
          w = jnp.exp(e)
          w_v[pl.ds(q * (L * L) + j * L, L)] = w
          plsc.addupdate_scatter(denp_v, [dfv[j] + iota16], w, mask=mask8)
        return c2

      lax.fori_loop(0, CA // L, q_body, 0)
      pltpu.sync_copy(w_v, wbuf.at[pl.ds(off * L, CA * L)])
      return carry

    lax.fori_loop(0, epc, chunk_body, 0)
    pltpu.sync_copy(denp_v, den_part.at[wid])

  return edge_stats


def _make_aggregate(n_pad, e_pad, d, heads):
  ept = e_pad // NW
  epc = ept // CB
  rpt = n_pad // NS
  nseg = d // L
  mesh = plsc.VectorSubcoreMesh(core_axis_name="c", subcore_axis_name="s")

  @functools.partial(
      pl.kernel,
      mesh=mesh,
      out_type=[
          jax.ShapeDtypeStruct((NC, n_pad, d), jnp.float32),
      ],
      scratch_types=[
          pltpu.VMEM((CB,), jnp.int32),
          pltpu.VMEM((CB,), jnp.int32),
          pltpu.VMEM((CB, d), jnp.float32),
          pltpu.VMEM((CB * L,), jnp.float32),
          pltpu.VMEM_SHARED((n_pad, d), jnp.float32),
          pltpu.SemaphoreType.DMA,
      ],
  )
  def aggregate(h, wbuf, srcA, dstA, out_part,
                sidx_v, didx_v, hrows_v, w_v,
                out_sh, sem):
    cid = lax.axis_index("c")
    sid = lax.axis_index("s")
    wid = sid * NC + cid

    _zero_vmem_rows(hrows_v, CB, d)
    zb = 128
    for t in range(rpt // zb):
      pltpu.sync_copy(hrows_v.at[pl.ds(0, zb), :],
                      out_sh.at[pl.ds(sid * rpt + t * zb, zb), :])
    plsc.subcore_barrier()

    base = wid * ept

    def chunk_body(c, carry):
      off = base + c * CB
      pltpu.sync_copy(srcA.at[pl.ds(off, CB)], sidx_v)
      pltpu.sync_copy(dstA.at[pl.ds(off, CB)], didx_v)
      d1 = pltpu.async_copy(h.at[sidx_v], hrows_v, sem)
      pltpu.sync_copy(wbuf.at[pl.ds(off * L, CB * L)], w_v)
      d1.wait()

      def q_body(q, c2):
        for j in range(L):
          k = q * L + j
          w = w_v[pl.ds(q * (L * L) + j * L, L)]
          for s in range(nseg):
            cj = w[s * heads // nseg]
            hrows_v[k, pl.ds(s * L, L)] = hrows_v[k, pl.ds(s * L, L)] * cj
        return c2

      lax.fori_loop(0, CB // L, q_body, 0)
      pltpu.sync_copy(hrows_v, out_sh.at[didx_v], add=True)
      return carry

    lax.fori_loop(0, epc, chunk_body, 0)
    plsc.subcore_barrier()
    pltpu.sync_copy(out_sh.at[pl.ds(sid * rpt, rpt), :],
                    out_part.at[cid, pl.ds(sid * rpt, rpt), :])

  return aggregate


def _build_A(a, heads, dout):
  hd = heads * dout
  A = jnp.zeros((hd, L), jnp.float32)
  rows = jnp.arange(hd)
  cols = jnp.repeat(jnp.arange(heads), dout)
  return A.at[rows, cols].set(a.reshape(-1))


def kernel(x, edge_index, W0, a_src0, a_dst0, b0, g0, be0,
           W1, a_src1, a_dst1, b1, g1, be1,
           W2, a_src2, a_dst2, b2, g2, be2):
  N, D = x.shape
  E = edge_index.shape[1]
  E2 = E + N
  e_pad = _cdiv(E2, NW * CB) * (NW * CB)
  n_pad = _cdiv(N + 1, 256) * 256

  sl = jnp.arange(N, dtype=jnp.int32)
  pad = e_pad - E2
  srcA = jnp.concatenate([edge_index[0].astype(jnp.int32), sl,
                          jnp.zeros((pad,), jnp.int32)])
  dstA = jnp.concatenate([edge_index[1].astype(jnp.int32), sl,
                          jnp.full((pad,), N, jnp.int32)])
  nch = e_pad // CA
  pkA = jnp.stack([
      (srcA // 8).reshape(nch, CA),
      (dstA // 8).reshape(nch, CA),
      ((srcA % 8) * L).reshape(nch, CA),
      ((dstA % 8) * L).reshape(nch, CA),
      (dstA * 8).reshape(nch, CA),
  ], axis=1).reshape(-1)

  exp8 = jnp.zeros((8, 128), jnp.float32).at[
      jnp.repeat(jnp.arange(8), 16), jnp.arange(128)].set(1.0)
  exp1 = jnp.zeros((8, 128), jnp.float32).at[0, :].set(1.0)

  edge_stats = _make_edge_stats(n_pad, e_pad)
  agg8 = _make_aggregate(n_pad, e_pad, D, 8)
  agg1 = _make_aggregate(n_pad, e_pad, D, 1)

  zpad = jnp.zeros((n_pad - N, L), jnp.float32)

  def layer(xin, W, a_src, a_dst, b, g, be, heads, act):
    dout = W.shape[1] // heads
    h, asrc, adst = _tc_prep(xin, W, _build_A(a_src, heads, dout),
                             _build_A(a_dst, heads, dout))
    stats_src = jnp.concatenate([asrc, zpad]).reshape(n_pad // 8, 128)
    stats_dst = jnp.concatenate([adst, zpad]).reshape(n_pad // 8, 128)
    wbuf, den_part = edge_stats(stats_src, stats_dst, pkA)
    invrep = _tc_invexp(den_part.reshape(NW, n_pad, 8),
                        exp8 if heads == 8 else exp1)
    agg = agg8 if heads == 8 else agg1
    (out_part,) = agg(h, wbuf, srcA, dstA)
    return _tc_post(out_part[0, :N], out_part[1, :N], invrep[:N], xin,
                    b, g, be, act)

  x1 = layer(x, W0, a_src0, a_dst0, b0, g0, be0, 8, True)
  x2 = layer(x1, W1, a_src1, a_dst1, b1, g1, be1, 8, True)
  x3 = layer(x2, W2, a_src2, a_dst2, b2, g2, be2, 1, False)
  return x3


# speedup vs baseline: 1.3144x; 1.3144x over previous
import functools

import jax
import jax.numpy as jnp
from jax import lax
from jax.experimental import pallas as pl
from jax.experimental.pallas import tpu as pltpu
from jax.experimental.pallas import tpu_sc as plsc

NC = 2
NS = 16
L = 16
NW = NC * NS
CA = 128
CB = 128
ROW_BLK = 2000


def _cdiv(a, b):
  return (a + b - 1) // b


def _prep_body(x_ref, w_ref, asrc_ref, adst_ref, h_ref, os_ref, od_ref):
  xb = x_ref[...]
  h = jnp.dot(xb, w_ref[...], preferred_element_type=jnp.float32,
              precision=lax.Precision.HIGHEST)
  h_ref[...] = h
  os_ref[...] = jnp.dot(h, asrc_ref[...], preferred_element_type=jnp.float32,
                        precision=lax.Precision.HIGHEST)
  od_ref[...] = jnp.dot(h, adst_ref[...], preferred_element_type=jnp.float32,
                        precision=lax.Precision.HIGHEST)


def _tc_prep(x, W, Asrc, Adst):
  N, D = x.shape
  grid = (N // ROW_BLK,)
  return pl.pallas_call(
      _prep_body,
      grid=grid,
      in_specs=[
          pl.BlockSpec((ROW_BLK, D), lambda i: (i, 0)),
          pl.BlockSpec(W.shape, lambda i: (0, 0)),
          pl.BlockSpec(Asrc.shape, lambda i: (0, 0)),
          pl.BlockSpec(Adst.shape, lambda i: (0, 0)),
      ],
      out_specs=[
          pl.BlockSpec((ROW_BLK, W.shape[1]), lambda i: (i, 0)),
          pl.BlockSpec((ROW_BLK, L), lambda i: (i, 0)),
          pl.BlockSpec((ROW_BLK, L), lambda i: (i, 0)),
      ],
      out_shape=[
          jax.ShapeDtypeStruct((N, W.shape[1]), jnp.float32),
          jax.ShapeDtypeStruct((N, L), jnp.float32),
          jax.ShapeDtypeStruct((N, L), jnp.float32),
      ],
  )(x, W, Asrc, Adst)


def _invden_body(p_ref, e_ref, inv_ref):
  s = jnp.sum(p_ref[...], axis=0)
  inv = 1.0 / (s + 1e-16)
  inv_ref[...] = jnp.dot(inv, e_ref[...], preferred_element_type=jnp.float32,
                         precision=lax.Precision.HIGHEST)


def _tc_invexp(parts, expand):
  _, R, H = parts.shape
  BR = 640
  grid = (R // BR,)
  return pl.pallas_call(
      _invden_body,
      grid=grid,
      in_specs=[
          pl.BlockSpec((NW, BR, H), lambda i: (0, i, 0)),
          pl.BlockSpec(expand.shape, lambda i: (0, 0)),
      ],
      out_specs=pl.BlockSpec((BR, 128), lambda i: (i, 0)),
      out_shape=jax.ShapeDtypeStruct((R, 128), jnp.float32),
  )(parts, expand)


def _post_body(p0_ref, p1_ref, inv_ref, res_ref, b_ref, g_ref, be_ref, o_ref,
               *, act):
  s = (p0_ref[...] + p1_ref[...]) * inv_ref[...] + b_ref[...]
  mu = jnp.mean(s, axis=-1, keepdims=True)
  var = jnp.mean((s - mu) ** 2, axis=-1, keepdims=True)
  y = (s - mu) / jnp.sqrt(var + 1e-5) * g_ref[...] + be_ref[...]
  if act:
    y = jnp.where(y >= 0, y, 0.2 * y) + res_ref[...]
  o_ref[...] = y


def _tc_post(p0, p1, inv, res, b, g, be, act):
  N, D = p0.shape
  grid = (N // ROW_BLK,)
  vec = pl.BlockSpec((1, D), lambda i: (0, 0))
  return pl.pallas_call(
      functools.partial(_post_body, act=act),
      grid=grid,
      in_specs=[
          pl.BlockSpec((ROW_BLK, D), lambda i: (i, 0)),
          pl.BlockSpec((ROW_BLK, D), lambda i: (i, 0)),
          pl.BlockSpec((ROW_BLK, D), lambda i: (i, 0)),
          pl.BlockSpec((ROW_BLK, D), lambda i: (i, 0)),
          vec, vec, vec,
      ],
      out_specs=pl.BlockSpec((ROW_BLK, D), lambda i: (i, 0)),
      out_shape=jax.ShapeDtypeStruct((N, D), jnp.float32),
  )(p0, p1, inv, res, b.reshape(1, D), g.reshape(1, D), be.reshape(1, D))


def _zero_vmem_rows(ref, nrows, ncols):
  nseg = ncols // L

  def body(k, c):
    for j in range(nseg):
      ref[k, pl.ds(j * L, L)] = jnp.zeros((L,), jnp.float32)
    return c

  lax.fori_loop(0, nrows, body, 0)


def _make_edge_stats(n_pad, e_pad):
  ept = e_pad // NW
  epc = ept // CA
  dn = n_pad * 8
  mesh = plsc.VectorSubcoreMesh(core_axis_name="c", subcore_axis_name="s")

  @functools.partial(
      pl.kernel,
      mesh=mesh,
      compiler_params=pltpu.CompilerParams(needs_layout_passes=False),
      out_type=[
          jax.ShapeDtypeStruct((e_pad * L,), jnp.float32),
          jax.ShapeDtypeStruct((NW, dn), jnp.float32),
      ],
      scratch_types=[
          pltpu.VMEM((5 * CA,), jnp.int32),
          pltpu.VMEM((CA, 128), jnp.float32),
          pltpu.VMEM((CA, 128), jnp.float32),
          pltpu.VMEM((CA * L,), jnp.float32),
          pltpu.VMEM((dn,), jnp.float32),
          pltpu.SemaphoreType.DMA,
      ],
  )
  def edge_stats(stats_src, stats_dst, pk,
                 wbuf, den_part,
                 pk_v, srows_v, drows_v, w_v,
                 denp_v, sem):
    cid = lax.axis_index("c")
    sid = lax.axis_index("s")
    wid = sid * NC + cid

    def zbody(i, c):
      denp_v[pl.ds(i * L, L)] = jnp.zeros((L,), jnp.float32)
      return c

    lax.fori_loop(0, dn // L, zbody, 0)

    iota16 = lax.iota(jnp.int32, 16)
    mask8 = iota16 < 8
    base = wid * ept

    def chunk_body(c, carry):
      off = base + c * CA
      pltpu.sync_copy(pk.at[pl.ds((wid * epc + c) * 5 * CA, 5 * CA)], pk_v)
      d1 = pltpu.async_copy(stats_src.at[pk_v.at[pl.ds(0, CA)]], srows_v, sem)
      d2 = pltpu.async_copy(stats_dst.at[pk_v.at[pl.ds(CA, CA)]], drows_v, sem)
      d1.wait()
      d2.wait()

      def q_body(q, c2):
        sov = pk_v[pl.ds(2 * CA + q * L, L)]
        dov = pk_v[pl.ds(3 * CA + q * L, L)]
        dfv = pk_v[pl.ds(4 * CA + q * L, L)]
        for j in range(L):
          k = q * L + j
          sv = srows_v[k, pl.ds(sov[j], L)]
          dv = drows_v[k, pl.ds(dov[j], L)]
          e = sv + dv
          e = jnp.where(e >= 0, e, e * 0.2)
          w = jnp.exp(e)
          w_v[pl.ds(q * (L * L) + j * L, L)] = w
          plsc.addupdate_scatter(denp_v, [dfv[j] + iota16], w, mask=mask8)
        return c2

      lax.fori_loop(0, CA // L, q_body, 0)
      pltpu.sync_copy(w_v, wbuf.at[pl.ds(off * L, CA * L)])
      return carry

    lax.fori_loop(0, epc, chunk_body, 0)
    pltpu.sync_copy(denp_v, den_part.at[wid])

  return edge_stats


def _make_aggregate(n_pad, e_pad, d, heads):
  ept = e_pad // NW
  epc = ept // CB
  rpt = n_pad // NS
  nseg = d // L
  mesh = plsc.VectorSubcoreMesh(core_axis_name="c", subcore_axis_name="s")

  @functools.partial(
      pl.kernel,
      mesh=mesh,
      out_type=[
          jax.ShapeDtypeStruct((NC, n_pad, d), jnp.float32),
      ],
      scratch_types=[
          pltpu.VMEM((CB,), jnp.int32),
          pltpu.VMEM((CB,), jnp.int32),
          pltpu.VMEM((CB, d), jnp.float32),
          pltpu.VMEM((CB * L,), jnp.float32),
          pltpu.VMEM_SHARED((n_pad, d), jnp.float32),
          pltpu.SemaphoreType.DMA,
      ],
  )
  def aggregate(h, wbuf, srcA, dstA, out_part,
                sidx_v, didx_v, hrows_v, w_v,
                out_sh, sem):
    cid = lax.axis_index("c")
    sid = lax.axis_index("s")
    wid = sid * NC + cid

    _zero_vmem_rows(hrows_v, CB, d)
    for t in range(rpt // CB):
      pltpu.sync_copy(hrows_v,
                      out_sh.at[pl.ds(sid * rpt + t * CB, CB), :])
    plsc.subcore_barrier()

    base = wid * ept

    def chunk_body(c, carry):
      off = base + c * CB
      pltpu.sync_copy(srcA.at[pl.ds(off, CB)], sidx_v)
      pltpu.sync_copy(dstA.at[pl.ds(off, CB)], didx_v)
      d1 = pltpu.async_copy(h.at[sidx_v], hrows_v, sem)
      pltpu.sync_copy(wbuf.at[pl.ds(off * L, CB * L)], w_v)
      d1.wait()

      def q_body(q, c2):
        for j in range(L):
          k = q * L + j
          w = w_v[pl.ds(q * (L * L) + j * L, L)]
          for s in range(nseg):
            cj = w[s * heads // nseg]
            hrows_v[k, pl.ds(s * L, L)] = hrows_v[k, pl.ds(s * L, L)] * cj
        return c2

      lax.fori_loop(0, CB // L, q_body, 0)
      pltpu.sync_copy(hrows_v, out_sh.at[didx_v], add=True)
      return carry

    lax.fori_loop(0, epc, chunk_body, 0)
    plsc.subcore_barrier()
    pltpu.sync_copy(out_sh.at[pl.ds(sid * rpt, rpt), :],
                    out_part.at[cid, pl.ds(sid * rpt, rpt), :])

  return aggregate


def _build_A(a, heads, dout):
  hd = heads * dout
  A = jnp.zeros((hd, L), jnp.float32)
  rows = jnp.arange(hd)
  cols = jnp.repeat(jnp.arange(heads), dout)
  return A.at[rows, cols].set(a.reshape(-1))


def kernel(x, edge_index, W0, a_src0, a_dst0, b0, g0, be0,
           W1, a_src1, a_dst1, b1, g1, be1,
           W2, a_src2, a_dst2, b2, g2, be2):
  N, D = x.shape
  E = edge_index.shape[1]
  E2 = E + N
  e_pad = _cdiv(E2, NW * CB) * (NW * CB)
  n_pad = _cdiv(N + 1, 256) * 256

  sl = jnp.arange(N, dtype=jnp.int32)
  pad = e_pad - E2
  srcA = jnp.concatenate([edge_index[0].astype(jnp.int32), sl,
                          jnp.zeros((pad,), jnp.int32)])
  dstA = jnp.concatenate([edge_index[1].astype(jnp.int32), sl,
                          jnp.full((pad,), N, jnp.int32)])
  nch = e_pad // CA
  pkA = jnp.stack([
      (srcA // 8).reshape(nch, CA),
      (dstA // 8).reshape(nch, CA),
      ((srcA % 8) * L).reshape(nch, CA),
      ((dstA % 8) * L).reshape(nch, CA),
      (dstA * 8).reshape(nch, CA),
  ], axis=1).reshape(-1)

  exp8 = jnp.zeros((8, 128), jnp.float32).at[
      jnp.repeat(jnp.arange(8), 16), jnp.arange(128)].set(1.0)
  exp1 = jnp.zeros((8, 128), jnp.float32).at[0, :].set(1.0)

  edge_stats = _make_edge_stats(n_pad, e_pad)
  agg8 = _make_aggregate(n_pad, e_pad, D, 8)
  agg1 = _make_aggregate(n_pad, e_pad, D, 1)

  zpad = jnp.zeros((n_pad - N, L), jnp.float32)

  def layer(xin, W, a_src, a_dst, b, g, be, heads, act):
    dout = W.shape[1] // heads
    h, asrc, adst = _tc_prep(xin, W, _build_A(a_src, heads, dout),
                             _build_A(a_dst, heads, dout))
    stats_src = jnp.concatenate([asrc, zpad]).reshape(n_pad // 8, 128)
    stats_dst = jnp.concatenate([adst, zpad]).reshape(n_pad // 8, 128)
    wbuf, den_part = edge_stats(stats_src, stats_dst, pkA)
    invrep = _tc_invexp(den_part.reshape(NW, n_pad, 8),
                        exp8 if heads == 8 else exp1)
    agg = agg8 if heads == 8 else agg1
    (out_part,) = agg(h, wbuf, srcA, dstA)
    return _tc_post(out_part[0, :N], out_part[1, :N], invrep[:N], xin,
                    b, g, be, act)

  x1 = layer(x, W0, a_src0, a_dst0, b0, g0, be0, 8, True)
  x2 = layer(x1, W1, a_src1, a_dst1, b1, g1, be1, 8, True)
  x3 = layer(x2, W2, a_src2, a_dst2, b2, g2, be2, 1, False)
  return x3
